# baseline (speedup 1.0000x reference)
import functools

import jax
import jax.numpy as jnp
from jax import lax
from jax.experimental import pallas as pl
from jax.experimental.pallas import tpu as pltpu
from jax.experimental.pallas import tpu_sc as plsc

_NC = 2
_NS = 16
_NW = _NC * _NS

_G = 128
_GS = 64
_KT = 30
_KS = 40
_NBT = 3
_NBS = 4


def kernel(tokens, W_E):
    B, S = tokens.shape
    V, D = W_E.shape
    N = B * S
    ng = N // (_NW * _G)
    assert ng * _G == _KT * _G + _KS * _GS
    nit = _KT // _NBT
    assert _KT == 3 * nit and _KS == 4 * nit

    idx3 = tokens.reshape(_NW, ng, _G).astype(jnp.int32)
    mesh = plsc.VectorSubcoreMesh(core_axis_name="c", subcore_axis_name="s")

    @functools.partial(
        pl.kernel,
        out_type=jax.ShapeDtypeStruct((N, D), jnp.float32),
        mesh=mesh,
        scratch_types=[
            pltpu.VMEM((ng, _G), jnp.int32),
            pltpu.VMEM((_NBT, _G, D), jnp.float32),
            pltpu.VMEM((_NBS, _GS, D), jnp.float32),
            pltpu.VMEM_SHARED((_NS, _NBS, _GS, D), jnp.float32),
            [pltpu.SemaphoreType.DMA] * _NBT,
            [pltpu.SemaphoreType.DMA] * _NBT,
            [pltpu.SemaphoreType.DMA] * _NBS,
            [pltpu.SemaphoreType.DMA] * _NBS,
            [pltpu.SemaphoreType.DMA] * _NBS,
        ],
    )
    def emb(idx_hbm, table_hbm, out_hbm, idx_v, trows, srows, slots,
            tgs, tws, sgs, sts, sws):
        wid = lax.axis_index("s") * _NC + lax.axis_index("c")
        sid = lax.axis_index("s")
        base = wid * (ng * _G)
        sbase = base + _KT * _G
        pltpu.sync_copy(idx_hbm.at[wid], idx_v)

        def tg(i, b):
            return pltpu.make_async_copy(
                table_hbm.at[idx_v.at[i]], trows.at[b], tgs[b])

        def tw(i, b):
            return pltpu.make_async_copy(
                trows.at[b], out_hbm.at[pl.ds(base + i * _G, _G)], tws[b])

        def sidx(j, h):
            return idx_v.at[_KT + j // 2, pl.ds(h * _GS, _GS)]

        def sg(j, h, b):
            return pltpu.make_async_copy(
                table_hbm.at[sidx(j, h)], srows.at[b], sgs[b])

        def st(b):
            return pltpu.make_async_copy(srows.at[b], slots.at[sid, b], sts[b])

        def sw(j, b):
            return pltpu.make_async_copy(
                slots.at[sid, b],
                out_hbm.at[pl.ds(sbase + j * _GS, _GS)], sws[b])

        def tvisit(i, b):
            tg(i, b).wait()
            tw(i, b).start()

            @pl.when(i >= 1)
            def _():
                tw(i - 1, (b - 1) % _NBT).wait()

            @pl.when(i + 2 < _KT)
            def _():
                tg(i + 2, (b + 2) % _NBT).start()

        def svisit(j, q):
            sg(j, q % 2, q).wait()

            @pl.when(j >= 4)
            def _():
                sw(j - 4, q).wait()

            st(q).start()

            @pl.when(j >= 1)
            def _():
                st((q - 1) % _NBS).wait()
                sw(j - 1, (q - 1) % _NBS).start()

            @pl.when(j + 3 < _KS)
            def _():
                sg(j + 3, (q + 3) % 2, (q + 3) % _NBS).start()

        for b in range(2):
            tg(b, b).start()
        for b in range(3):
            sg(b, b % 2, b).start()

        def body(u, carry):
            for v in range(3):
                tvisit(3 * u + v, v)
            for q in range(4):
                svisit(4 * u + q, q)
            return carry

        lax.fori_loop(0, nit, body, 0)

        tw(_KT - 1, (_KT - 1) % _NBT).wait()
        st((_KS - 1) % _NBS).wait()
        sw(_KS - 1, (_KS - 1) % _NBS).start()
        for j in range(_KS - 4, _KS):
            sw(j, j % _NBS).wait()

    out = emb(idx3, W_E)
    return out.reshape(B, S, D)
